# baseline (speedup 1.0000x reference)
import jax
import jax.numpy as jnp
from jax.experimental import pallas as pl
from jax.experimental.pallas import tpu as pltpu


_BLK = 512


def _maxlen_kernel(m_ref, ml_ref):
    counts = jnp.sum((m_ref[...] != 0).astype(jnp.int32), axis=1)
    ml_ref[0] = jnp.maximum(jnp.max(counts), 1)


def _trim_kernel(ml_ref, x_ref, v_ref, m_ref, xo_ref, vo_ref, mo_ref):
    j = pl.program_id(0)
    ml = ml_ref[0]
    col = j * _BLK + jax.lax.broadcasted_iota(jnp.int32, (1, _BLK), 1)
    keep = col < ml
    xo_ref[...] = jnp.where(keep, x_ref[...], 0.0)
    vo_ref[...] = jnp.where(keep, v_ref[...], 0.0)
    mo_ref[...] = jnp.where(keep & (m_ref[...] != 0), 1, 0).astype(jnp.int32)


def kernel(x, v, mask):
    B, C, P = x.shape
    CV = v.shape[1]
    xr = x.reshape(B * C, P)
    vr = v.reshape(B * CV, P)
    mr = mask.reshape(B, P)

    ml = pl.pallas_call(
        _maxlen_kernel,
        out_shape=jax.ShapeDtypeStruct((1,), jnp.int32),
        out_specs=pl.BlockSpec(memory_space=pltpu.SMEM),
    )(mr)

    nblk = P // _BLK

    def in_map(j, ml_s):
        return (0, jnp.minimum(j, jnp.maximum(ml_s[0] - 1, 0) // _BLK))

    def out_map(j, ml_s):
        return (0, j)

    grid_spec = pltpu.PrefetchScalarGridSpec(
        num_scalar_prefetch=1,
        grid=(nblk,),
        in_specs=[
            pl.BlockSpec((B * C, _BLK), in_map),
            pl.BlockSpec((B * CV, _BLK), in_map),
            pl.BlockSpec((B, _BLK), in_map),
        ],
        out_specs=[
            pl.BlockSpec((B * C, _BLK), out_map),
            pl.BlockSpec((B * CV, _BLK), out_map),
            pl.BlockSpec((B, _BLK), out_map),
        ],
    )
    xo, vo, mo = pl.pallas_call(
        _trim_kernel,
        grid_spec=grid_spec,
        out_shape=[
            jax.ShapeDtypeStruct((B * C, P), jnp.float32),
            jax.ShapeDtypeStruct((B * CV, P), jnp.float32),
            jax.ShapeDtypeStruct((B, P), jnp.int32),
        ],
    )(ml, xr, vr, mr)
    return (xo.reshape(B, C, P), vo.reshape(B, CV, P), mo.reshape(B, 1, P))
